# baseline (speedup 1.0000x reference)
import jax
import jax.numpy as jnp
from jax import lax
from jax.experimental import pallas as pl
from jax.experimental.pallas import tpu as pltpu
from jax.experimental.pallas import tpu_sc as plsc

_B = 4096
_EMB = 64
_NC, _NS = 2, 16
_NW = _NC * _NS
_BPW = _B // _NW


def _sc_gather_body(table_hbm, idx_hbm, out_hbm, rows_v, idx_v, sem):
    wid = lax.axis_index("s") * _NC + lax.axis_index("c")
    base = wid * _BPW
    pltpu.sync_copy(idx_hbm.at[pl.ds(base, _BPW)], idx_v)

    def issue(g, carry):
        v = idx_v[pl.ds(g * 16, 16)]
        for k in range(16):
            pltpu.async_copy(table_hbm.at[pl.ds(v[k], 1)],
                             rows_v.at[pl.ds(g * 16 + k, 1)], sem)
        return carry

    lax.fori_loop(0, _BPW // 16, issue, 0)
    pltpu.make_async_copy(table_hbm.at[pl.ds(0, _BPW)], rows_v, sem).wait()
    pltpu.sync_copy(rows_v, out_hbm.at[pl.ds(base, _BPW)])


def _sc_gather(table, idx):
    return pl.kernel(
        _sc_gather_body,
        mesh=plsc.VectorSubcoreMesh(core_axis_name="c", subcore_axis_name="s"),
        out_type=jax.ShapeDtypeStruct((_B, _EMB), jnp.float32),
        scratch_types=[
            pltpu.VMEM((_BPW, _EMB), jnp.float32),
            pltpu.VMEM((_BPW,), jnp.int32),
            pltpu.SemaphoreType.DMA,
        ],
    )(table, idx)


def _mlp_body(id_emb, sec, stg, reg, n0, n1, n2, n3, n4, n5,
              sec_tT, stg_t, reg_tT,
              w1, b1, g1, be1, w2, b2, g2, be2, out):
    f32 = jnp.float32
    dim0 = (((0,), (0,)), ((), ()))

    def onehotT(idx_ref, n):
        iota = lax.broadcasted_iota(jnp.int32, (n, _B), 0)
        return (iota == idx_ref[...][None, :]).astype(f32)

    numT = jnp.stack([n0[...], n1[...], n2[...], n3[...], n4[...], n5[...]],
                     axis=0)
    feats = jnp.concatenate(
        [onehotT(sec, 32), onehotT(stg, 16), onehotT(reg, 24), numT],
        axis=0)
    g_small = jnp.concatenate(
        [lax.dot_general(sec_tT[...], w1[pl.ds(_EMB, 16), :], dim0,
                         preferred_element_type=f32),
         jnp.dot(stg_t[...], w1[pl.ds(_EMB + 16, 16), :],
                 preferred_element_type=f32),
         lax.dot_general(reg_tT[...], w1[pl.ds(_EMB + 32, 16), :], dim0,
                         preferred_element_type=f32),
         w1[pl.ds(_EMB + 48, 6), :]],
        axis=0)

    h = (jnp.dot(id_emb[...].astype(jnp.bfloat16),
                 w1[pl.ds(0, _EMB), :].astype(jnp.bfloat16),
                 preferred_element_type=f32)
         + lax.dot_general(feats, g_small, dim0, preferred_element_type=f32)
         + b1[...][None, :])
    h = jnp.maximum(h, 0.0)
    mu = jnp.mean(h, axis=0, keepdims=True)
    var = jnp.mean((h - mu) ** 2, axis=0, keepdims=True)
    h = g1[...][None, :] * (h - mu) * lax.rsqrt(var + 1e-5) + be1[...][None, :]

    h2 = (jnp.dot(h.astype(jnp.bfloat16), w2[...].astype(jnp.bfloat16),
                  preferred_element_type=f32) + b2[...][None, :])
    h2 = jnp.maximum(h2, 0.0)
    mu2 = jnp.mean(h2, axis=0, keepdims=True)
    var2 = jnp.mean((h2 - mu2) ** 2, axis=0, keepdims=True)
    h2 = (g2[...][None, :] * (h2 - mu2) * lax.rsqrt(var2 + 1e-5)
          + be2[...][None, :])

    nrm = jnp.sqrt(jnp.sum(h2 * h2, axis=1, keepdims=True))
    out[...] = h2 / jnp.maximum(nrm, 1e-12)


def kernel(id, sector, stage, region, deal_size, revenue_multiple,
           growth_rate, profitability, team_experience, market_size,
           deal_table, sector_table, stage_table, region_table,
           W1, b1, g1, beta1, W2, b2, g2, beta2):
    id_emb = _sc_gather(deal_table, id.astype(jnp.int32))
    return pl.pallas_call(
        _mlp_body,
        out_shape=jax.ShapeDtypeStruct((_B, W2.shape[1]), jnp.float32),
    )(id_emb, sector.astype(jnp.int32), stage.astype(jnp.int32),
      region.astype(jnp.int32), deal_size, revenue_multiple, growth_rate,
      profitability, team_experience, market_size, sector_table.T, stage_table,
      region_table.T, W1, b1, g1, beta1, W2, b2, g2, beta2)
